# baseline (speedup 1.0000x reference)
import functools

import jax
import jax.numpy as jnp
from jax import lax
from jax.experimental import pallas as pl
from jax.experimental.pallas import tpu as pltpu
from jax.experimental.pallas import tpu_sc as plsc

BS = 1024
NV = 501
NPAD = 512
CH = 128
D = 64
L = 16
NC = 2
NS = 16
NW = NC * NS
BPW = BS // NW
NCH = NPAD // CH
NT = BPW * NCH


def _trig_body(day_ref, w1_ref, w2_ref, dr_ref, di_ref):
    dayv = day_ref[:]
    dr_ref[:] = jnp.cos(w2_ref[:] * dayv)
    di_ref[:] = jnp.sin(w1_ref[:] * dayv)


def _trig(day, w1, w2):
    return pl.pallas_call(
        _trig_body,
        out_shape=(jax.ShapeDtypeStruct((BS, D), jnp.float32),
                   jax.ShapeDtypeStruct((BS, D), jnp.float32)),
    )(day.reshape(BS, 1), w1.reshape(1, D), w2.reshape(1, D))


def _loss_body(sc_ref, out_ref):
    s = sc_ref[:]
    col = lax.broadcasted_iota(jnp.int32, (BS, NPAD), 1)
    s = jnp.where(col < NV, s, -jnp.inf)
    m = jnp.max(s, axis=1, keepdims=True)
    e = jnp.exp(s - m)
    lse = jnp.log(jnp.sum(e, axis=1, keepdims=True)) + m
    loss2d = lse - sc_ref[:, 0:1]
    out_ref[:] = jnp.mean(loss2d).reshape(1, 1)


def _loss(scores):
    return pl.pallas_call(
        _loss_body,
        out_shape=jax.ShapeDtypeStruct((1, 1), jnp.float32),
    )(scores)


_mesh = plsc.VectorSubcoreMesh(core_axis_name="c", subcore_axis_name="s")


@functools.partial(
    pl.kernel,
    mesh=_mesh,
    compiler_params=pltpu.CompilerParams(
        needs_layout_passes=False, use_tc_tiling_on_sc=False),
    out_type=jax.ShapeDtypeStruct((BS, NPAD), jnp.float32),
    scratch_types=[
        pltpu.VMEM((BPW, NCH, CH), jnp.int32),
        pltpu.VMEM((2, CH, D), jnp.float32),
        pltpu.VMEM((2, CH, D), jnp.float32),
        pltpu.VMEM((BPW,), jnp.int32),
        pltpu.VMEM((BPW,), jnp.int32),
        pltpu.VMEM((BPW, D), jnp.float32),
        pltpu.VMEM((BPW, D), jnp.float32),
        pltpu.VMEM((BPW, D), jnp.float32),
        pltpu.VMEM((BPW, D), jnp.float32),
        pltpu.VMEM((BPW, D), jnp.float32),
        pltpu.VMEM((BPW, D), jnp.float32),
        pltpu.VMEM((BPW, D), jnp.float32),
        pltpu.VMEM((BPW, D), jnp.float32),
        pltpu.VMEM((BPW, NPAD), jnp.float32),
        pltpu.SemaphoreType.DMA,
        pltpu.SemaphoreType.DMA,
        pltpu.SemaphoreType.DMA,
        pltpu.SemaphoreType.DMA,
        pltpu.SemaphoreType.DMA,
    ],
)
def _score(ids_hbm, sub_hbm, rel_hbm, dreal_hbm, dimg_hbm,
           embEr_hbm, embEi_hbm, embRr_hbm, embRi_hbm,
           out_hbm,
           ids_v, er_buf, ei_buf, sub_i, rel_i, sr, si, rr, ri, dr, di,
           ar_all, ai_all, scores_v,
           s_er0, s_ei0, s_er1, s_ei1, s_misc):
    wid = lax.axis_index("s") * NC + lax.axis_index("c")
    b0 = wid * BPW

    pltpu.sync_copy(ids_hbm.at[pl.ds(b0, BPW)], ids_v)
    pltpu.sync_copy(sub_hbm.at[pl.ds(b0, BPW)], sub_i)
    pltpu.sync_copy(rel_hbm.at[pl.ds(b0, BPW)], rel_i)
    pltpu.sync_copy(dreal_hbm.at[pl.ds(b0, BPW)], dr)
    pltpu.sync_copy(dimg_hbm.at[pl.ds(b0, BPW)], di)
    pltpu.async_copy(embEr_hbm.at[sub_i], sr, s_misc).wait()
    pltpu.async_copy(embEi_hbm.at[sub_i], si, s_misc).wait()
    pltpu.async_copy(embRr_hbm.at[rel_i], rr, s_misc).wait()
    pltpu.async_copy(embRi_hbm.at[rel_i], ri, s_misc).wait()

    def a_body(bl, carry):
        for g in range(D // L):
            slg = pl.ds(g * L, L)
            c = dr[bl, slg]
            s = di[bl, slg]
            svr = sr[bl, slg]
            svi = si[bl, slg]
            ar_all[bl, slg] = svr * c - svi * s + rr[bl, slg]
            ai_all[bl, slg] = svr * s + svi * c + ri[bl, slg]
        return carry
    lax.fori_loop(0, BPW, a_body, 0)

    sems = ((s_er0, s_ei0), (s_er1, s_ei1))

    def fire(t, p):
        bl = lax.div(t, NCH)
        ci = lax.rem(t, NCH)
        idx = ids_v.at[bl, ci]
        pltpu.async_copy(embEr_hbm.at[idx], er_buf.at[p], sems[p][0])
        pltpu.async_copy(embEi_hbm.at[idx], ei_buf.at[p], sems[p][1])

    def wait_for(t, p):
        bl = lax.div(t, NCH)
        ci = lax.rem(t, NCH)
        idx = ids_v.at[bl, ci]
        pltpu.make_async_copy(embEr_hbm.at[idx], er_buf.at[p], sems[p][0]).wait()
        pltpu.make_async_copy(embEi_hbm.at[idx], ei_buf.at[p], sems[p][1]).wait()

    fire(jnp.int32(0), 0)
    lane = lax.iota(jnp.int32, L)
    m0 = lane == 0

    def step(t, p):
        @pl.when(t + 1 < NT)
        def _():
            fire(t + 1, 1 - p)

        wait_for(t, p)
        bl = lax.div(t, NCH)
        ci = lax.rem(t, NCH)
        ebase = ci * CH
        bl_vec = jnp.broadcast_to(bl, (L,))
        cs = [dr[bl, pl.ds(g * L, L)] for g in range(D // L)]
        ss = [di[bl, pl.ds(g * L, L)] for g in range(D // L)]
        ars = [ar_all[bl, pl.ds(g * L, L)] for g in range(D // L)]
        ais = [ai_all[bl, pl.ds(g * L, L)] for g in range(D // L)]

        def e_body(j, carry):
            acc = jnp.zeros((L,), jnp.float32)
            for g in range(D // L):
                slg = pl.ds(g * L, L)
                er = er_buf[p, j, slg]
                ei = ei_buf[p, j, slg]
                vr = ars[g] - er * cs[g] + ei * ss[g]
                vi = ais[g] + er * ss[g] + ei * cs[g]
                acc = acc + jnp.abs(vr) + jnp.abs(vi)
            sv = jnp.broadcast_to(jnp.sum(acc), (L,))
            pos_vec = jnp.broadcast_to(ebase + j, (L,))
            plsc.store_scatter(scores_v, [bl_vec, pos_vec], sv, mask=m0)
            return carry
        lax.fori_loop(0, CH, e_body, 0, unroll=8)

    def outer(tt, carry):
        step(2 * tt, 0)
        step(2 * tt + 1, 1)
        return carry
    lax.fori_loop(0, NT // 2, outer, 0)

    pltpu.sync_copy(scores_v, out_hbm.at[pl.ds(b0, BPW)])


def kernel(sub, rel, obj, year, month, day, neg, emb_E_real, emb_E_img,
           emb_R_real, emb_R_img, w1, w2):
    dreal, dimg = _trig(day, w1, w2)
    ids = jnp.concatenate([obj[:, None], neg], axis=1)
    ids = jnp.pad(ids, ((0, 0), (0, NPAD - NV)))
    ids = ids.reshape(BS, NPAD // CH, CH)
    scores = _score(ids, sub, rel, dreal, dimg,
                    emb_E_real, emb_E_img, emb_R_real, emb_R_img)
    return _loss(scores)[0, 0]


# speedup vs baseline: 1.0043x; 1.0043x over previous
import functools

import jax
import jax.numpy as jnp
from jax import lax
from jax.experimental import pallas as pl
from jax.experimental.pallas import tpu as pltpu
from jax.experimental.pallas import tpu_sc as plsc

BS = 1024
NV = 501
NPAD = 512
CH = 128
D = 64
L = 16
NC = 2
NS = 16
NW = NC * NS
BPW = BS // NW
NCH = NPAD // CH
NT = BPW * NCH


def _trig_body(day_ref, w1_ref, w2_ref, dr_ref, di_ref):
    dayv = day_ref[:]
    dr_ref[:] = jnp.cos(w2_ref[:] * dayv)
    di_ref[:] = jnp.sin(w1_ref[:] * dayv)


def _trig(day, w1, w2):
    return pl.pallas_call(
        _trig_body,
        out_shape=(jax.ShapeDtypeStruct((BS, D), jnp.float32),
                   jax.ShapeDtypeStruct((BS, D), jnp.float32)),
    )(day.reshape(BS, 1), w1.reshape(1, D), w2.reshape(1, D))


def _loss_body(sc_ref, out_ref):
    s = sc_ref[:]
    col = lax.broadcasted_iota(jnp.int32, (BS, NPAD), 1)
    s = jnp.where(col < NV, s, -jnp.inf)
    m = jnp.max(s, axis=1, keepdims=True)
    e = jnp.exp(s - m)
    lse = jnp.log(jnp.sum(e, axis=1, keepdims=True)) + m
    loss2d = lse - sc_ref[:, 0:1]
    out_ref[:] = jnp.mean(loss2d).reshape(1, 1)


def _loss(scores):
    return pl.pallas_call(
        _loss_body,
        out_shape=jax.ShapeDtypeStruct((1, 1), jnp.float32),
    )(scores)


_mesh = plsc.VectorSubcoreMesh(core_axis_name="c", subcore_axis_name="s")


@functools.partial(
    pl.kernel,
    mesh=_mesh,
    compiler_params=pltpu.CompilerParams(
        needs_layout_passes=False, use_tc_tiling_on_sc=False),
    out_type=jax.ShapeDtypeStruct((BS, NPAD), jnp.float32),
    scratch_types=[
        pltpu.VMEM((BPW, NCH, CH), jnp.int32),
        pltpu.VMEM((4, CH, D), jnp.float32),
        pltpu.VMEM((4, CH, D), jnp.float32),
        pltpu.VMEM((BPW,), jnp.int32),
        pltpu.VMEM((BPW,), jnp.int32),
        pltpu.VMEM((BPW, D), jnp.float32),
        pltpu.VMEM((BPW, D), jnp.float32),
        pltpu.VMEM((BPW, D), jnp.float32),
        pltpu.VMEM((BPW, D), jnp.float32),
        pltpu.VMEM((BPW, D), jnp.float32),
        pltpu.VMEM((BPW, D), jnp.float32),
        pltpu.VMEM((BPW, D), jnp.float32),
        pltpu.VMEM((BPW, D), jnp.float32),
        pltpu.VMEM((BPW, NPAD), jnp.float32),
        pltpu.SemaphoreType.DMA,
        pltpu.SemaphoreType.DMA,
        pltpu.SemaphoreType.DMA,
        pltpu.SemaphoreType.DMA,
        pltpu.SemaphoreType.DMA,
        pltpu.SemaphoreType.DMA,
        pltpu.SemaphoreType.DMA,
        pltpu.SemaphoreType.DMA,
        pltpu.SemaphoreType.DMA,
    ],
)
def _score(ids_hbm, sub_hbm, rel_hbm, dreal_hbm, dimg_hbm,
           embEr_hbm, embEi_hbm, embRr_hbm, embRi_hbm,
           out_hbm,
           ids_v, er_buf, ei_buf, sub_i, rel_i, sr, si, rr, ri, dr, di,
           ar_all, ai_all, scores_v,
           s_er0, s_ei0, s_er1, s_ei1, s_er2, s_ei2, s_er3, s_ei3, s_misc):
    wid = lax.axis_index("s") * NC + lax.axis_index("c")
    b0 = wid * BPW

    pltpu.sync_copy(ids_hbm.at[pl.ds(b0, BPW)], ids_v)
    pltpu.sync_copy(sub_hbm.at[pl.ds(b0, BPW)], sub_i)
    pltpu.sync_copy(rel_hbm.at[pl.ds(b0, BPW)], rel_i)
    pltpu.sync_copy(dreal_hbm.at[pl.ds(b0, BPW)], dr)
    pltpu.sync_copy(dimg_hbm.at[pl.ds(b0, BPW)], di)
    pltpu.async_copy(embEr_hbm.at[sub_i], sr, s_misc).wait()
    pltpu.async_copy(embEi_hbm.at[sub_i], si, s_misc).wait()
    pltpu.async_copy(embRr_hbm.at[rel_i], rr, s_misc).wait()
    pltpu.async_copy(embRi_hbm.at[rel_i], ri, s_misc).wait()

    def a_body(bl, carry):
        for g in range(D // L):
            slg = pl.ds(g * L, L)
            c = dr[bl, slg]
            s = di[bl, slg]
            svr = sr[bl, slg]
            svi = si[bl, slg]
            ar_all[bl, slg] = svr * c - svi * s + rr[bl, slg]
            ai_all[bl, slg] = svr * s + svi * c + ri[bl, slg]
        return carry
    lax.fori_loop(0, BPW, a_body, 0)

    sems = ((s_er0, s_ei0), (s_er1, s_ei1), (s_er2, s_ei2), (s_er3, s_ei3))

    def fire(t, p):
        bl = lax.div(t, NCH)
        ci = lax.rem(t, NCH)
        idx = ids_v.at[bl, ci]
        pltpu.async_copy(embEr_hbm.at[idx], er_buf.at[p], sems[p][0])
        pltpu.async_copy(embEi_hbm.at[idx], ei_buf.at[p], sems[p][1])

    def wait_for(t, p):
        bl = lax.div(t, NCH)
        ci = lax.rem(t, NCH)
        idx = ids_v.at[bl, ci]
        pltpu.make_async_copy(embEr_hbm.at[idx], er_buf.at[p], sems[p][0]).wait()
        pltpu.make_async_copy(embEi_hbm.at[idx], ei_buf.at[p], sems[p][1]).wait()

    fire(jnp.int32(0), 0)
    fire(jnp.int32(1), 1)
    fire(jnp.int32(2), 2)
    lane = lax.iota(jnp.int32, L)
    m0 = lane == 0

    def step(t, p):
        @pl.when(t + 3 < NT)
        def _():
            fire(t + 3, (p + 3) % 4)

        wait_for(t, p)
        bl = lax.div(t, NCH)
        ci = lax.rem(t, NCH)
        ebase = ci * CH
        bl_vec = jnp.broadcast_to(bl, (L,))
        cs = [dr[bl, pl.ds(g * L, L)] for g in range(D // L)]
        ss = [di[bl, pl.ds(g * L, L)] for g in range(D // L)]
        ars = [ar_all[bl, pl.ds(g * L, L)] for g in range(D // L)]
        ais = [ai_all[bl, pl.ds(g * L, L)] for g in range(D // L)]

        def e_body(j, carry):
            acc = jnp.zeros((L,), jnp.float32)
            for g in range(D // L):
                slg = pl.ds(g * L, L)
                er = er_buf[p, j, slg]
                ei = ei_buf[p, j, slg]
                vr = ars[g] - er * cs[g] + ei * ss[g]
                vi = ais[g] + er * ss[g] + ei * cs[g]
                acc = acc + jnp.abs(vr) + jnp.abs(vi)
            sv = jnp.broadcast_to(jnp.sum(acc), (L,))
            pos_vec = jnp.broadcast_to(ebase + j, (L,))
            plsc.store_scatter(scores_v, [bl_vec, pos_vec], sv, mask=m0)
            return carry
        lax.fori_loop(0, CH, e_body, 0, unroll=8)

    def outer(tt, carry):
        step(4 * tt, 0)
        step(4 * tt + 1, 1)
        step(4 * tt + 2, 2)
        step(4 * tt + 3, 3)
        return carry
    lax.fori_loop(0, NT // 4, outer, 0)

    pltpu.sync_copy(scores_v, out_hbm.at[pl.ds(b0, BPW)])


def kernel(sub, rel, obj, year, month, day, neg, emb_E_real, emb_E_img,
           emb_R_real, emb_R_img, w1, w2):
    dreal, dimg = _trig(day, w1, w2)
    ids = jnp.concatenate([obj[:, None], neg], axis=1)
    ids = jnp.pad(ids, ((0, 0), (0, NPAD - NV)))
    ids = ids.reshape(BS, NPAD // CH, CH)
    scores = _score(ids, sub, rel, dreal, dimg,
                    emb_E_real, emb_E_img, emb_R_real, emb_R_img)
    return _loss(scores)[0, 0]
